# Initial kernel scaffold; baseline (speedup 1.0000x reference)
import jax
import jax.numpy as jnp
from jax.experimental import pallas as pl


def kernel(tokens, embedding_table):
    raise NotImplementedError("write your pallas kernel here")


# speedup vs baseline: 4.8242x; 4.8242x over previous
import functools

import jax
import jax.numpy as jnp
from jax import lax
from jax.experimental import pallas as pl
from jax.experimental.pallas import tpu as pltpu
from jax.experimental.pallas import tpu_sc as plsc

_NC = 2
_NS = 16
_NW = _NC * _NS

_B = 4096 * 200
_D = 64
_CH = 1024
_BPW = _B // _NW
_NCHUNK = _BPW // _CH

_mesh = plsc.VectorSubcoreMesh(core_axis_name="c", subcore_axis_name="s")


@functools.partial(
    pl.kernel,
    mesh=_mesh,
    out_type=jax.ShapeDtypeStruct((_B, _D), jnp.float32),
    scratch_types=[
        pltpu.VMEM((_CH,), jnp.int32),
        pltpu.VMEM((_CH, _D), jnp.float32),
        pltpu.SemaphoreType.DMA,
    ],
    compiler_params=pltpu.CompilerParams(use_tc_tiling_on_sc=False),
)
def _gather_kernel(idx_hbm, table_hbm, out_hbm, idx_v, rows_v, sem):
    wid = lax.axis_index("s") * _NC + lax.axis_index("c")
    base = wid * _BPW

    def body(i, carry):
        off = base + i * _CH
        pltpu.sync_copy(idx_hbm.at[pl.ds(off, _CH)], idx_v)
        pltpu.async_copy(table_hbm.at[idx_v], rows_v, sem).wait()
        pltpu.sync_copy(rows_v, out_hbm.at[pl.ds(off, _CH)])
        return carry

    lax.fori_loop(0, _NCHUNK, body, 0)


def kernel(tokens, embedding_table):
    idx = tokens.reshape(-1).astype(jnp.int32)
    out = _gather_kernel(idx, embedding_table)
    return out.reshape(tokens.shape + (embedding_table.shape[1],))


# speedup vs baseline: 4.9681x; 1.0298x over previous
import functools

import jax
import jax.numpy as jnp
from jax import lax
from jax.experimental import pallas as pl
from jax.experimental.pallas import tpu as pltpu
from jax.experimental.pallas import tpu_sc as plsc

_NC = 2
_NS = 16
_NW = _NC * _NS

_B = 4096 * 200
_D = 64
_CH = 400
_NBUF = 4
_BPW = _B // _NW
_NCHUNK = _BPW // _CH

_mesh = plsc.VectorSubcoreMesh(core_axis_name="c", subcore_axis_name="s")


@functools.partial(
    pl.kernel,
    mesh=_mesh,
    out_type=jax.ShapeDtypeStruct((_B, _D), jnp.float32),
    scratch_types=[
        pltpu.VMEM((_NCHUNK, _CH), jnp.int32),
        [pltpu.VMEM((_CH, _D), jnp.float32) for _ in range(_NBUF)],
        [pltpu.SemaphoreType.DMA for _ in range(_NBUF)],
        [pltpu.SemaphoreType.DMA for _ in range(_NBUF)],
    ],
    compiler_params=pltpu.CompilerParams(use_tc_tiling_on_sc=False),
)
def _gather_kernel(idx_hbm, table_hbm, out_hbm, idx_v, rows, gsem, osem):
    wid = lax.axis_index("s") * _NC + lax.axis_index("c")
    base = wid * _BPW

    pltpu.sync_copy(idx_hbm.at[wid], idx_v)

    for b in range(_NBUF):
        pltpu.async_copy(table_hbm.at[idx_v.at[b]], rows[b], gsem[b])

    def outer(k, carry):
        i0 = k * _NBUF
        for b in range(_NBUF):
            i = i0 + b
            out_slice = out_hbm.at[pl.ds(base + i * _CH, _CH)]
            pltpu.make_async_copy(table_hbm.at[idx_v.at[i]], rows[b], gsem[b]).wait()
            pltpu.async_copy(rows[b], out_slice, osem[b])
            j = i + _NBUF

            @pl.when(j < _NCHUNK)
            def _():
                pltpu.make_async_copy(rows[b], out_slice, osem[b]).wait()
                pltpu.async_copy(table_hbm.at[idx_v.at[j]], rows[b], gsem[b])
        return carry

    lax.fori_loop(0, _NCHUNK // _NBUF, outer, 0)

    for b in range(_NBUF):
        i = _NCHUNK - _NBUF + b
        out_slice = out_hbm.at[pl.ds(base + i * _CH, _CH)]
        pltpu.make_async_copy(rows[b], out_slice, osem[b]).wait()


def kernel(tokens, embedding_table):
    idx = tokens.reshape(_NW, _NCHUNK, _CH).astype(jnp.int32)
    out = _gather_kernel(idx, embedding_table)
    return out.reshape(tokens.shape + (embedding_table.shape[1],))
